# baseline (speedup 1.0000x reference)
import functools

import jax
import jax.numpy as jnp
from jax import lax
from jax.experimental import pallas as pl
from jax.experimental.pallas import tpu as pltpu
from jax.experimental.pallas import tpu_sc as plsc

_NBUF = 4


def _gather_flat(obs_flat, table, n_workers, chunk):
    n = obs_flat.shape[0]
    d = table.shape[1]
    per_w = n // n_workers
    steps = per_w // chunk
    assert steps % _NBUF == 0 and steps >= 2 * _NBUF
    mesh = plsc.VectorSubcoreMesh(core_axis_name="c", subcore_axis_name="s")

    @functools.partial(
        pl.kernel,
        mesh=mesh,
        out_type=jax.ShapeDtypeStruct((n, d), jnp.float32),
        scratch_types=[
            pltpu.VMEM((_NBUF, chunk), jnp.int32),
            pltpu.VMEM((_NBUF, chunk, d), jnp.float32),
        ]
        + [pltpu.SemaphoreType.DMA] * (3 * _NBUF),
        compiler_params=pltpu.CompilerParams(use_tc_tiling_on_sc=False),
    )
    def run(obs_hbm, table_hbm, out_hbm, idx_v, rows_v, *sems):
        gat = sems[:_NBUF]
        out = sems[_NBUF : 2 * _NBUF]
        isem = sems[2 * _NBUF :]
        info = plsc.get_sparse_core_info()
        nc = info.num_cores
        wid = lax.axis_index("s") * nc + lax.axis_index("c")
        wbase = wid * per_w

        def idx_cp(g, b):
            g = g % steps
            return pltpu.make_async_copy(
                obs_hbm.at[pl.ds(wbase + g * chunk, chunk)],
                idx_v.at[b],
                isem[b],
            )

        def gather_cp(g, b):
            del g
            return pltpu.make_async_copy(
                table_hbm.at[idx_v.at[b]],
                rows_v.at[b],
                gat[b],
            )

        def store_cp(g, b):
            return pltpu.make_async_copy(
                rows_v.at[b],
                out_hbm.at[pl.ds(wbase + g * chunk, chunk)],
                out[b],
            )

        for b in range(_NBUF):
            idx_cp(b, b).start()
        for b in range(_NBUF):
            idx_cp(b, b).wait()
            gather_cp(b, b).start()
        for b in range(2):
            gather_cp(b, b).wait()
            store_cp(b, b).start()
            idx_cp(b + _NBUF, b).start()

        def body(k, carry):
            g0 = _NBUF * k
            for b in range(_NBUF):
                g = g0 + b
                store_cp(g - _NBUF, b).wait()
                idx_cp(g, b).wait()
                gather_cp(g, b).start()
                b2 = (b + 2) % _NBUF
                gather_cp(g - 2, b2).wait()
                store_cp(g - 2, b2).start()
                idx_cp(g + 2, b2).start()
            return carry

        lax.fori_loop(1, steps // _NBUF, body, 0)

        for g in (steps - 2, steps - 1):
            b = g % _NBUF
            gather_cp(g, b).wait()
            store_cp(g, b).start()
        for b in range(2):
            idx_cp(steps + b, b).wait()
        for b in range(_NBUF):
            store_cp(steps - _NBUF + b, b).wait()

    return run(obs_flat, table)


def kernel(obs, table):
    b, f = obs.shape
    d = table.shape[1]
    n = b * f
    obs_flat = obs.reshape(n).astype(jnp.int32)
    out = _gather_flat(obs_flat, table, n_workers=32, chunk=800)
    return out.reshape(b, f * d)


# speedup vs baseline: 1.0004x; 1.0004x over previous
import functools

import jax
import jax.numpy as jnp
from jax import lax
from jax.experimental import pallas as pl
from jax.experimental.pallas import tpu as pltpu
from jax.experimental.pallas import tpu_sc as plsc

_NBUF = 4


def _gather_flat(obs_flat, table, n_workers, chunk):
    n = obs_flat.shape[0]
    d = table.shape[1]
    per_w = n // n_workers
    steps = per_w // chunk
    assert steps % _NBUF == 0 and steps >= 2 * _NBUF
    mesh = plsc.VectorSubcoreMesh(core_axis_name="c", subcore_axis_name="s")

    @functools.partial(
        pl.kernel,
        mesh=mesh,
        out_type=jax.ShapeDtypeStruct((n, d), jnp.float32),
        scratch_types=[
            pltpu.VMEM((_NBUF, chunk), jnp.int32),
            pltpu.VMEM((_NBUF, chunk, d), jnp.float32),
        ]
        + [pltpu.SemaphoreType.DMA] * (3 * _NBUF),
        compiler_params=pltpu.CompilerParams(use_tc_tiling_on_sc=False),
    )
    def run(obs_hbm, table_hbm, out_hbm, idx_v, rows_v, *sems):
        gat = sems[:_NBUF]
        out = sems[_NBUF : 2 * _NBUF]
        isem = sems[2 * _NBUF :]
        info = plsc.get_sparse_core_info()
        nc = info.num_cores
        wid = lax.axis_index("s") * nc + lax.axis_index("c")
        wbase = wid * per_w

        def idx_cp(g, b):
            g = g % steps
            return pltpu.make_async_copy(
                obs_hbm.at[pl.ds((g * n_workers + wid) * chunk, chunk)],
                idx_v.at[b],
                isem[b],
            )

        def gather_cp(g, b):
            del g
            return pltpu.make_async_copy(
                table_hbm.at[idx_v.at[b]],
                rows_v.at[b],
                gat[b],
            )

        def store_cp(g, b):
            return pltpu.make_async_copy(
                rows_v.at[b],
                out_hbm.at[pl.ds((g * n_workers + wid) * chunk, chunk)],
                out[b],
            )

        for b in range(_NBUF):
            idx_cp(b, b).start()
        for b in range(_NBUF):
            idx_cp(b, b).wait()
            gather_cp(b, b).start()
        for b in range(2):
            gather_cp(b, b).wait()
            store_cp(b, b).start()
            idx_cp(b + _NBUF, b).start()

        def body(k, carry):
            g0 = _NBUF * k
            for b in range(_NBUF):
                g = g0 + b
                store_cp(g - _NBUF, b).wait()
                idx_cp(g, b).wait()
                gather_cp(g, b).start()
                b2 = (b + 2) % _NBUF
                gather_cp(g - 2, b2).wait()
                store_cp(g - 2, b2).start()
                idx_cp(g + 2, b2).start()
            return carry

        lax.fori_loop(1, steps // _NBUF, body, 0)

        for g in (steps - 2, steps - 1):
            b = g % _NBUF
            gather_cp(g, b).wait()
            store_cp(g, b).start()
        for b in range(2):
            idx_cp(steps + b, b).wait()
        for b in range(_NBUF):
            store_cp(steps - _NBUF + b, b).wait()

    return run(obs_flat, table)


def kernel(obs, table):
    b, f = obs.shape
    d = table.shape[1]
    n = b * f
    obs_flat = obs.reshape(n).astype(jnp.int32)
    out = _gather_flat(obs_flat, table, n_workers=32, chunk=800)
    return out.reshape(b, f * d)


# speedup vs baseline: 1.0010x; 1.0006x over previous
import functools

import jax
import jax.numpy as jnp
from jax import lax
from jax.experimental import pallas as pl
from jax.experimental.pallas import tpu as pltpu
from jax.experimental.pallas import tpu_sc as plsc

_NBUF = 4


def _gather_flat(obs_flat, table, n_workers, chunk):
    n = obs_flat.shape[0]
    d = table.shape[1]
    per_w = n // n_workers
    steps = per_w // chunk
    assert steps % _NBUF == 0 and steps >= 2 * _NBUF
    mesh = plsc.VectorSubcoreMesh(core_axis_name="c", subcore_axis_name="s")

    @functools.partial(
        pl.kernel,
        mesh=mesh,
        out_type=jax.ShapeDtypeStruct((n, d), jnp.float32),
        scratch_types=[
            pltpu.VMEM((per_w,), jnp.int32),
            pltpu.VMEM((_NBUF, chunk, d), jnp.float32),
        ]
        + [pltpu.SemaphoreType.DMA] * (2 * _NBUF),
        compiler_params=pltpu.CompilerParams(use_tc_tiling_on_sc=False),
    )
    def run(obs_hbm, table_hbm, out_hbm, idx_v, rows_v, *sems):
        gat = sems[:_NBUF]
        out = sems[_NBUF:]
        info = plsc.get_sparse_core_info()
        nc = info.num_cores
        wid = lax.axis_index("s") * nc + lax.axis_index("c")
        wbase = wid * per_w

        pltpu.sync_copy(obs_hbm.at[pl.ds(wbase, per_w)], idx_v)

        def gather_cp(g, b):
            return pltpu.make_async_copy(
                table_hbm.at[idx_v.at[pl.ds(g * chunk, chunk)]],
                rows_v.at[b],
                gat[b],
            )

        def store_cp(g, b):
            return pltpu.make_async_copy(
                rows_v.at[b],
                out_hbm.at[pl.ds(wbase + g * chunk, chunk)],
                out[b],
            )

        for b in range(_NBUF):
            gather_cp(b, b).start()
        for b in range(2):
            gather_cp(b, b).wait()
            store_cp(b, b).start()

        def body(k, carry):
            g0 = _NBUF * k
            for b in range(_NBUF):
                g = g0 + b
                store_cp(g - _NBUF, b).wait()
                gather_cp(g, b).start()
                b2 = (b + 2) % _NBUF
                gather_cp(g - 2, b2).wait()
                store_cp(g - 2, b2).start()
            return carry

        lax.fori_loop(1, steps // _NBUF, body, 0)

        for g in (steps - 2, steps - 1):
            b = g % _NBUF
            gather_cp(g, b).wait()
            store_cp(g, b).start()
        for b in range(_NBUF):
            store_cp(steps - _NBUF + b, b).wait()

    return run(obs_flat, table)


def kernel(obs, table):
    b, f = obs.shape
    d = table.shape[1]
    n = b * f
    obs_flat = obs.reshape(n).astype(jnp.int32)
    out = _gather_flat(obs_flat, table, n_workers=32, chunk=512)
    return out.reshape(b, f * d)
